# Initial kernel scaffold; baseline (speedup 1.0000x reference)
import jax
import jax.numpy as jnp
from jax.experimental import pallas as pl


def kernel(z, edge_index, W_in, b_in, W_msg, b_msg, W_upd, b_upd, W_out, b_out):
    raise NotImplementedError("write your pallas kernel here")


# speedup vs baseline: 7.4795x; 7.4795x over previous
import functools

import jax
import jax.numpy as jnp
from jax import lax
from jax.experimental import pallas as pl
from jax.experimental.pallas import tpu as pltpu
from jax.experimental.pallas import tpu_sc as plsc

N = 10000
H = 128
E = 320000
ROUNDS = 3

NC = 2
NS = 16
NW = NC * NS
K = 125
EPW = E // NW
NCH = EPW // K
NP = 10240
RPT = NP // NS

@functools.cache
def _make_sc_edge_aggregate():
    mesh = plsc.VectorSubcoreMesh(core_axis_name="c", subcore_axis_name="s")

    @functools.partial(
        pl.kernel,
        out_type=jax.ShapeDtypeStruct((NC, NP, H), jnp.float32),
        mesh=mesh,
        scratch_types=[
            pltpu.VMEM((NCH, K), jnp.int32),
            pltpu.VMEM((NCH, K), jnp.int32),
            pltpu.VMEM((K, H), jnp.float32),
            pltpu.VMEM_SHARED((NP, H), jnp.float32),
            pltpu.SemaphoreType.DMA,
        ],
    )
    def sc_edge_aggregate(msg_hbm, src_hbm, dst_hbm, zero_hbm, out_hbm,
                          src_v, dst_v, rows_v, acc_sh, sem):
        cid = lax.axis_index("c")
        sid = lax.axis_index("s")
        wid = cid * NS + sid
        pltpu.sync_copy(zero_hbm.at[pl.ds(sid * RPT, RPT)],
                        acc_sh.at[pl.ds(sid * RPT, RPT)])
        pltpu.sync_copy(src_hbm.at[pl.ds(wid * NCH, NCH)], src_v)
        pltpu.sync_copy(dst_hbm.at[pl.ds(wid * NCH, NCH)], dst_v)
        plsc.subcore_barrier()

        def chunk(j, carry):
            pltpu.async_copy(msg_hbm.at[src_v.at[j]], rows_v, sem).wait()
            pltpu.sync_copy(rows_v, acc_sh.at[dst_v.at[j]], add=True)
            return carry

        lax.fori_loop(0, NCH, chunk, 0)
        plsc.subcore_barrier()
        pltpu.sync_copy(acc_sh.at[pl.ds(sid * RPT, RPT)],
                        out_hbm.at[cid, pl.ds(sid * RPT, RPT)])

    return sc_edge_aggregate


BM = 1000


def _lin_relu_body(x_ref, w_ref, b_ref, o_ref):
    o_ref[...] = jnp.maximum(
        jnp.dot(x_ref[...], w_ref[...], preferred_element_type=jnp.float32)
        + b_ref[...], 0.0)


def _lin_relu(x, w, b):
    return pl.pallas_call(
        _lin_relu_body,
        grid=(N // BM,),
        in_specs=[
            pl.BlockSpec((BM, H), lambda i: (i, 0)),
            pl.BlockSpec((H, H), lambda i: (0, 0)),
            pl.BlockSpec((1, H), lambda i: (0, 0)),
        ],
        out_specs=pl.BlockSpec((BM, H), lambda i: (i, 0)),
        out_shape=jax.ShapeDtypeStruct((N, H), jnp.float32),
    )(x, w, b.reshape(1, H))


def _update_body(s_ref, p0_ref, p1_ref, w_ref, b_ref, o_ref):
    agg = p0_ref[...] + p1_ref[...]
    o_ref[...] = s_ref[...] + jnp.maximum(
        jnp.dot(agg, w_ref[...], preferred_element_type=jnp.float32)
        + b_ref[...], 0.0)


def _update(s, p0, p1, w, b):
    return pl.pallas_call(
        _update_body,
        grid=(N // BM,),
        in_specs=[
            pl.BlockSpec((BM, H), lambda i: (i, 0)),
            pl.BlockSpec((BM, H), lambda i: (i, 0)),
            pl.BlockSpec((BM, H), lambda i: (i, 0)),
            pl.BlockSpec((H, H), lambda i: (0, 0)),
            pl.BlockSpec((1, H), lambda i: (0, 0)),
        ],
        out_specs=pl.BlockSpec((BM, H), lambda i: (i, 0)),
        out_shape=jax.ShapeDtypeStruct((N, H), jnp.float32),
    )(s, p0, p1, w, b.reshape(1, H))


def _final_body(x_ref, w_ref, b_ref, o_ref):
    o_ref[...] = (
        jnp.dot(x_ref[...], w_ref[...], preferred_element_type=jnp.float32)
        + b_ref[...])


def _final(x, w, b):
    return pl.pallas_call(
        _final_body,
        grid=(N // BM,),
        in_specs=[
            pl.BlockSpec((BM, H), lambda i: (i, 0)),
            pl.BlockSpec((H, 128), lambda i: (0, 0)),
            pl.BlockSpec((1, 128), lambda i: (0, 0)),
        ],
        out_specs=pl.BlockSpec((BM, 128), lambda i: (i, 0)),
        out_shape=jax.ShapeDtypeStruct((N, 128), jnp.float32),
    )(x, w, b.reshape(1, 128))


def kernel(z, edge_index, W_in, b_in, W_msg, b_msg, W_upd, b_upd, W_out, b_out):
    src = edge_index[0].reshape(E // K, K)
    dst = edge_index[1].reshape(E // K, K)
    zeros = jnp.zeros((NP, H), jnp.float32)
    state = _lin_relu(z, W_in, b_in)
    for r in range(ROUNDS):
        message = _lin_relu(state, W_msg[r], b_msg[r])
        p = _make_sc_edge_aggregate()(message, src, dst, zeros)
        state = _update(state, p[0], p[1], W_upd[r], b_upd[r])
    w_pad = jnp.zeros((H, 128), jnp.float32).at[:, : W_out.shape[1]].set(W_out)
    b_pad = jnp.zeros((128,), jnp.float32).at[: b_out.shape[0]].set(b_out)
    out = _final(state, w_pad, b_pad)
    return out[:, : W_out.shape[1]]


# speedup vs baseline: 11.3524x; 1.5178x over previous
import functools

import jax
import jax.numpy as jnp
from jax import lax
from jax.experimental import pallas as pl
from jax.experimental.pallas import tpu as pltpu
from jax.experimental.pallas import tpu_sc as plsc

N = 10000
H = 128
E = 320000
ROUNDS = 3

NC = 2
NS = 16
NW = NC * NS
K = 125
EPW = E // NW
NCH = EPW // K
NP = 10240
RPT = NP // NS
NH = 2
CPH = NCH // NH


@functools.cache
def _make_sc_edge_aggregate():
    mesh = plsc.VectorSubcoreMesh(core_axis_name="c", subcore_axis_name="s")

    @functools.partial(
        pl.kernel,
        out_type=jax.ShapeDtypeStruct((NC, NP, H), jnp.float32),
        mesh=mesh,
        scratch_types=[
            pltpu.VMEM((CPH, K), jnp.int32),
            pltpu.VMEM((CPH, K), jnp.int32),
            pltpu.VMEM((K, H), jnp.float32),
            pltpu.VMEM((K, H), jnp.float32),
            pltpu.VMEM_SHARED((NP, H), jnp.float32),
            pltpu.SemaphoreType.DMA,
            pltpu.SemaphoreType.DMA,
        ],
    )
    def sc_edge_aggregate(msg_hbm, src_hbm, dst_hbm, zero_hbm, out_hbm,
                          src_v, dst_v, rows0, rows1, acc_sh, sem0, sem1):
        cid = lax.axis_index("c")
        sid = lax.axis_index("s")
        wid = cid * NS + sid

        def gather(j, rows, sem):
            return pltpu.make_async_copy(msg_hbm.at[src_v.at[j]], rows, sem)

        pltpu.sync_copy(zero_hbm.at[pl.ds(sid * RPT, RPT)],
                        acc_sh.at[pl.ds(sid * RPT, RPT)])
        plsc.subcore_barrier()

        def two_chunks(i, carry):
            j0 = 2 * i
            gather(j0 + 1, rows1, sem1).start()
            gather(j0, rows0, sem0).wait()
            pltpu.sync_copy(rows0, acc_sh.at[dst_v.at[j0]], add=True)

            @pl.when(j0 + 2 < CPH)
            def _():
                gather(j0 + 2, rows0, sem0).start()

            gather(j0 + 1, rows1, sem1).wait()
            pltpu.sync_copy(rows1, acc_sh.at[dst_v.at[j0 + 1]], add=True)
            return carry

        for h in range(NH):
            base = wid * NCH + h * CPH
            pltpu.sync_copy(src_hbm.at[pl.ds(base, CPH)], src_v)
            pltpu.sync_copy(dst_hbm.at[pl.ds(base, CPH)], dst_v)
            gather(0, rows0, sem0).start()
            lax.fori_loop(0, CPH // 2, two_chunks, 0)
        plsc.subcore_barrier()
        pltpu.sync_copy(acc_sh.at[pl.ds(sid * RPT, RPT)],
                        out_hbm.at[cid, pl.ds(sid * RPT, RPT)])

    return sc_edge_aggregate


BM = 1000


def _relu_mm(x, w, b):
    return jnp.maximum(
        jnp.dot(x, w, preferred_element_type=jnp.float32) + b, 0.0)


def _in_msg_body(z_ref, wi_ref, bi_ref, wm_ref, bm_ref, s_ref, m_ref):
    s = _relu_mm(z_ref[...], wi_ref[...], bi_ref[...])
    s_ref[...] = s
    m_ref[...] = _relu_mm(s, wm_ref[...], bm_ref[...])


def _in_msg(z, wi, bi, wm, bm):
    blk = pl.BlockSpec((BM, H), lambda i: (i, 0))
    wblk = pl.BlockSpec((H, H), lambda i: (0, 0))
    bblk = pl.BlockSpec((1, H), lambda i: (0, 0))
    return pl.pallas_call(
        _in_msg_body,
        grid=(N // BM,),
        in_specs=[blk, wblk, bblk, wblk, bblk],
        out_specs=[blk, blk],
        out_shape=[jax.ShapeDtypeStruct((N, H), jnp.float32)] * 2,
    )(z, wi, bi.reshape(1, H), wm, bm.reshape(1, H))


def _upd_msg_body(s_ref, p0_ref, p1_ref, wu_ref, bu_ref, wm_ref, bm_ref,
                  s_out_ref, m_ref):
    agg = p0_ref[...] + p1_ref[...]
    s = s_ref[...] + _relu_mm(agg, wu_ref[...], bu_ref[...])
    s_out_ref[...] = s
    m_ref[...] = _relu_mm(s, wm_ref[...], bm_ref[...])


def _upd_msg(s, p0, p1, wu, bu, wm, bm):
    blk = pl.BlockSpec((BM, H), lambda i: (i, 0))
    wblk = pl.BlockSpec((H, H), lambda i: (0, 0))
    bblk = pl.BlockSpec((1, H), lambda i: (0, 0))
    return pl.pallas_call(
        _upd_msg_body,
        grid=(N // BM,),
        in_specs=[blk, blk, blk, wblk, bblk, wblk, bblk],
        out_specs=[blk, blk],
        out_shape=[jax.ShapeDtypeStruct((N, H), jnp.float32)] * 2,
    )(s, p0, p1, wu, bu.reshape(1, H), wm, bm.reshape(1, H))


def _upd_out_body(s_ref, p0_ref, p1_ref, wu_ref, bu_ref, wo_ref, bo_ref,
                  o_ref):
    agg = p0_ref[...] + p1_ref[...]
    s = s_ref[...] + _relu_mm(agg, wu_ref[...], bu_ref[...])
    o_ref[...] = (
        jnp.dot(s, wo_ref[...], preferred_element_type=jnp.float32)
        + bo_ref[...])


def _upd_out(s, p0, p1, wu, bu, wo, bo):
    blk = pl.BlockSpec((BM, H), lambda i: (i, 0))
    wblk = pl.BlockSpec((H, H), lambda i: (0, 0))
    bblk = pl.BlockSpec((1, H), lambda i: (0, 0))
    return pl.pallas_call(
        _upd_out_body,
        grid=(N // BM,),
        in_specs=[blk, blk, blk, wblk, bblk,
                  pl.BlockSpec((H, 128), lambda i: (0, 0)),
                  pl.BlockSpec((1, 128), lambda i: (0, 0))],
        out_specs=pl.BlockSpec((BM, 128), lambda i: (i, 0)),
        out_shape=jax.ShapeDtypeStruct((N, 128), jnp.float32),
    )(s, p0, p1, wu, bu.reshape(1, H), wo, bo.reshape(1, 128))


def kernel(z, edge_index, W_in, b_in, W_msg, b_msg, W_upd, b_upd, W_out, b_out):
    src = edge_index[0].reshape(E // K, K)
    dst = edge_index[1].reshape(E // K, K)
    zeros = jnp.zeros((NP, H), jnp.float32)
    sc = _make_sc_edge_aggregate()

    state, message = _in_msg(z, W_in, b_in, W_msg[0], b_msg[0])
    for r in range(ROUNDS - 1):
        p = sc(message, src, dst, zeros)
        state, message = _upd_msg(state, p[0], p[1], W_upd[r], b_upd[r],
                                  W_msg[r + 1], b_msg[r + 1])
    p = sc(message, src, dst, zeros)
    w_pad = jnp.zeros((H, 128), jnp.float32).at[:, : W_out.shape[1]].set(W_out)
    b_pad = jnp.zeros((128,), jnp.float32).at[: b_out.shape[0]].set(b_out)
    out = _upd_out(state, p[0], p[1], W_upd[ROUNDS - 1], b_upd[ROUNDS - 1],
                   w_pad, b_pad)
    return out[:, : W_out.shape[1]]
